# baseline (speedup 1.0000x reference)
import functools

import jax
import jax.numpy as jnp
from jax import lax
from jax.experimental import pallas as pl
from jax.experimental.pallas import tpu as pltpu
from jax.experimental.pallas import tpu_sc as plsc

BATCH = 16384
EMBED_DIM = 32
HIDDEN_DIM = 64
FEAT_PAD = 8

_NC = 2
_NS = 16
_NW = _NC * _NS
_B_PER_W = BATCH // _NW


def _sc_gather(user_table, item_table, user_idx, item_idx):
    mesh = plsc.VectorSubcoreMesh(core_axis_name="c", subcore_axis_name="s")

    @functools.partial(
        pl.kernel,
        mesh=mesh,
        compiler_params=pltpu.CompilerParams(use_tc_tiling_on_sc=False),
        out_type=[
            jax.ShapeDtypeStruct((BATCH, EMBED_DIM), jnp.float32),
            jax.ShapeDtypeStruct((BATCH, EMBED_DIM), jnp.float32),
        ],
        scratch_types=[
            pltpu.VMEM((_B_PER_W,), jnp.int32),
            pltpu.VMEM((_B_PER_W, EMBED_DIM), jnp.float32),
            pltpu.VMEM((_B_PER_W,), jnp.int32),
            pltpu.VMEM((_B_PER_W, EMBED_DIM), jnp.float32),
            pltpu.SemaphoreType.DMA,
            pltpu.SemaphoreType.DMA,
        ],
    )
    def gather_kernel(ut_hbm, it_hbm, uidx_hbm, iidx_hbm, uout_hbm, iout_hbm,
                      uidx_v, urows_v, iidx_v, irows_v, sem_u, sem_i):
        wid = lax.axis_index("s") * _NC + lax.axis_index("c")
        base = wid * _B_PER_W
        pltpu.sync_copy(uidx_hbm.at[pl.ds(base, _B_PER_W)], uidx_v)
        pltpu.sync_copy(iidx_hbm.at[pl.ds(base, _B_PER_W)], iidx_v)
        cu = pltpu.async_copy(ut_hbm.at[uidx_v], urows_v, sem_u)
        ci = pltpu.async_copy(it_hbm.at[iidx_v], irows_v, sem_i)
        cu.wait()
        pltpu.sync_copy(urows_v, uout_hbm.at[pl.ds(base, _B_PER_W)])
        ci.wait()
        pltpu.sync_copy(irows_v, iout_hbm.at[pl.ds(base, _B_PER_W)])

    return gather_kernel(user_table, item_table, user_idx, item_idx)


def _mlp_body(ue_ref, ie_ref, f_ref, w1u_ref, w1i_ref, w1f_ref, b1_ref,
              w2_ref, b2_ref, out_ref):
    h = (
        jnp.dot(ue_ref[...], w1u_ref[...], preferred_element_type=jnp.float32)
        + jnp.dot(ie_ref[...], w1i_ref[...], preferred_element_type=jnp.float32)
        + jnp.dot(f_ref[...], w1f_ref[...], preferred_element_type=jnp.float32)
        + b1_ref[...]
    )
    h = jnp.maximum(h, 0.0)
    out_ref[...] = jnp.sum(h * w2_ref[...], axis=1, keepdims=True) + b2_ref[...]


def _tc_mlp(user_emb, item_emb, features_p, w1u, w1i, w1f, b1, w2r, b2):
    block = 2048
    grid = (BATCH // block,)
    const = lambda i: (0, 0)
    return pl.pallas_call(
        _mlp_body,
        grid=grid,
        in_specs=[
            pl.BlockSpec((block, EMBED_DIM), lambda i: (i, 0)),
            pl.BlockSpec((block, EMBED_DIM), lambda i: (i, 0)),
            pl.BlockSpec((block, FEAT_PAD), lambda i: (i, 0)),
            pl.BlockSpec((EMBED_DIM, HIDDEN_DIM), const),
            pl.BlockSpec((EMBED_DIM, HIDDEN_DIM), const),
            pl.BlockSpec((FEAT_PAD, HIDDEN_DIM), const),
            pl.BlockSpec((1, HIDDEN_DIM), const),
            pl.BlockSpec((1, HIDDEN_DIM), const),
            pl.BlockSpec((1, 1), const),
        ],
        out_specs=pl.BlockSpec((block, 1), lambda i: (i, 0)),
        out_shape=jax.ShapeDtypeStruct((BATCH, 1), jnp.float32),
    )(user_emb, item_emb, features_p, w1u, w1i, w1f, b1, w2r, b2)


def kernel(user, item, features, user_table, item_table, W1, b1, W2, b2):
    user = user.astype(jnp.int32)
    item = item.astype(jnp.int32)
    user_emb, item_emb = _sc_gather(user_table, item_table, user, item)

    features_p = jnp.pad(features, ((0, 0), (0, FEAT_PAD - features.shape[1])))
    w1u = W1[:EMBED_DIM]
    w1i = W1[EMBED_DIM:2 * EMBED_DIM]
    w1f = jnp.pad(W1[2 * EMBED_DIM:], ((0, FEAT_PAD - 3), (0, 0)))
    b1r = b1.reshape(1, HIDDEN_DIM)
    w2r = W2.reshape(1, HIDDEN_DIM)
    b2r = b2.reshape(1, 1)

    out = _tc_mlp(user_emb, item_emb, features_p, w1u, w1i, w1f, b1r, w2r, b2r)
    return out.reshape(BATCH)
